# baseline (speedup 1.0000x reference)
import functools

import jax
import jax.numpy as jnp
from jax import lax
from jax.experimental import pallas as pl
from jax.experimental.pallas import tpu as pltpu
from jax.experimental.pallas import tpu_sc as plsc

_D = 64
_B = 4096
_L = 200
_N = _B * _L
_NC = 2
_NS = 16
_NW = _NC * _NS
_PER_W = _N // _NW
_CHUNK = 128
_NCHUNK = _PER_W // _CHUNK
_NBUF = 8
_LOOK = 4

_mesh = plsc.VectorSubcoreMesh(core_axis_name="c", subcore_axis_name="s")


@functools.partial(
    pl.kernel,
    mesh=_mesh,
    compiler_params=pltpu.CompilerParams(use_tc_tiling_on_sc=False),
    out_type=jax.ShapeDtypeStruct((_N, _D), jnp.float32),
    scratch_types=[
        pltpu.VMEM((_NCHUNK, _CHUNK), jnp.int32),
        [pltpu.VMEM((_CHUNK, _D), jnp.float32) for _ in range(_NBUF)],
        [pltpu.SemaphoreType.DMA for _ in range(_NBUF)],
        [pltpu.SemaphoreType.DMA for _ in range(_NBUF)],
    ],
)
def _embedding_gather(w_hbm, idx_hbm, out_hbm, idx_v, bufs, gsems, osems):
    wid = lax.axis_index("s") * _NC + lax.axis_index("c")
    base = wid * _PER_W

    pltpu.sync_copy(idx_hbm.at[wid], idx_v)

    def gather_cp(c, b):
        return pltpu.make_async_copy(w_hbm.at[idx_v.at[c]], bufs[b], gsems[b])

    def out_cp(c, b):
        dst = out_hbm.at[pl.ds(base + c * _CHUNK, _CHUNK)]
        return pltpu.make_async_copy(bufs[b], dst, osems[b])

    for c in range(_LOOK):
        gather_cp(c, c % _NBUF).start()

    for c in range(_LOOK):
        b = c % _NBUF
        gather_cp(c, b).wait()
        out_cp(c, b).start()
        b2 = (c + _LOOK) % _NBUF
        gather_cp(c + _LOOK, b2).start()

    steady = _NCHUNK - 2 * _LOOK
    groups = steady // _NBUF

    @pl.loop(0, groups)
    def _steady(s):
        for j in range(_NBUF):
            c = _LOOK + s * _NBUF + j
            b = (_LOOK + j) % _NBUF
            gather_cp(c, b).wait()
            out_cp(c, b).start()
            b2 = j % _NBUF
            out_cp(c - _LOOK, b2).wait()
            gather_cp(c + _LOOK, b2).start()

    for c in range(_NCHUNK - _LOOK, _NCHUNK):
        b = c % _NBUF
        gather_cp(c, b).wait()
        out_cp(c, b).start()
        out_cp(c - _LOOK, (c + _LOOK) % _NBUF).wait()

    for c in range(_NCHUNK - _LOOK, _NCHUNK):
        out_cp(c, c % _NBUF).wait()


def kernel(x, w):
    idx = x.reshape(_NW, _NCHUNK, _CHUNK)
    out = _embedding_gather(w, idx)
    return out.reshape(_B, _L, _D)


# speedup vs baseline: 1.0039x; 1.0039x over previous
import functools

import jax
import jax.numpy as jnp
from jax import lax
from jax.experimental import pallas as pl
from jax.experimental.pallas import tpu as pltpu
from jax.experimental.pallas import tpu_sc as plsc

_D = 64
_B = 4096
_L = 200
_N = _B * _L
_NC = 2
_NS = 16
_NW = _NC * _NS
_PER_W = _N // _NW
_CHUNK = 256
_NCHUNK = _PER_W // _CHUNK
_NBUF = 4
_LOOK = 2

_mesh = plsc.VectorSubcoreMesh(core_axis_name="c", subcore_axis_name="s")


@functools.partial(
    pl.kernel,
    mesh=_mesh,
    compiler_params=pltpu.CompilerParams(use_tc_tiling_on_sc=False),
    out_type=jax.ShapeDtypeStruct((_N, _D), jnp.float32),
    scratch_types=[
        pltpu.VMEM((_NCHUNK, _CHUNK), jnp.int32),
        [pltpu.VMEM((_CHUNK, _D), jnp.float32) for _ in range(_NBUF)],
        [pltpu.SemaphoreType.DMA for _ in range(_NBUF)],
        [pltpu.SemaphoreType.DMA for _ in range(_NBUF)],
    ],
)
def _embedding_gather(w_hbm, idx_hbm, out_hbm, idx_v, bufs, gsems, osems):
    wid = lax.axis_index("s") * _NC + lax.axis_index("c")
    base = wid * _PER_W

    pltpu.sync_copy(idx_hbm.at[wid], idx_v)

    def gather_cp(c, b):
        return pltpu.make_async_copy(w_hbm.at[idx_v.at[c]], bufs[b], gsems[b])

    def out_cp(c, b):
        dst = out_hbm.at[pl.ds(base + c * _CHUNK, _CHUNK)]
        return pltpu.make_async_copy(bufs[b], dst, osems[b])

    for c in range(_LOOK):
        gather_cp(c, c % _NBUF).start()

    for c in range(_LOOK):
        b = c % _NBUF
        gather_cp(c, b).wait()
        out_cp(c, b).start()
        b2 = (c + _LOOK) % _NBUF
        gather_cp(c + _LOOK, b2).start()

    steady = _NCHUNK - 2 * _LOOK
    groups = steady // _NBUF

    @pl.loop(0, groups)
    def _steady(s):
        for j in range(_NBUF):
            c = _LOOK + s * _NBUF + j
            b = (_LOOK + j) % _NBUF
            gather_cp(c, b).wait()
            out_cp(c, b).start()
            b2 = j % _NBUF
            out_cp(c - _LOOK, b2).wait()
            gather_cp(c + _LOOK, b2).start()

    for c in range(_NCHUNK - _LOOK, _NCHUNK):
        b = c % _NBUF
        gather_cp(c, b).wait()
        out_cp(c, b).start()
        out_cp(c - _LOOK, (c + _LOOK) % _NBUF).wait()

    for c in range(_NCHUNK - _LOOK, _NCHUNK):
        out_cp(c, c % _NBUF).wait()


def kernel(x, w):
    idx = x.reshape(_NW, _NCHUNK, _CHUNK)
    out = _embedding_gather(w, idx)
    return out.reshape(_B, _L, _D)
